# baseline (speedup 1.0000x reference)
import functools

import jax
import jax.numpy as jnp
import numpy as np
from jax import lax
from jax.experimental import pallas as pl
from jax.experimental.pallas import tpu as pltpu
from jax.experimental.pallas import tpu_sc as plsc

_B, _D, _S, _K = 4096, 256, 64, 64
_BB = 64
_R = _BB * _S

_NW = 32
_SPW = _B // _NW
_CG = 4


def _dot(x, w, prec=jax.lax.Precision.DEFAULT):
    return jax.lax.dot_general(
        x, w, (((1,), (0,)), ((), ())),
        precision=prec, preferred_element_type=jnp.float32)


def _dotT(x, w):
    return jax.lax.dot_general(
        x, w, (((1,), (1,)), ((), ())),
        precision=jax.lax.Precision.HIGHEST,
        preferred_element_type=jnp.float32)


def _nv_body(hidden_ref, wv_ref, bv_ref, nv_ref):
    nv_ref[...] = _dotT(hidden_ref[...], wv_ref[...]) + bv_ref[...]


def _project_nv(hidden, Wv, bv):
    nbb = 512
    return pl.pallas_call(
        _nv_body,
        grid=(_B // nbb,),
        in_specs=[
            pl.BlockSpec((nbb, _D), lambda i: (i, 0)),
            pl.BlockSpec((_D, _D), lambda i: (0, 0)),
            pl.BlockSpec((1, _D), lambda i: (0, 0)),
        ],
        out_specs=pl.BlockSpec((nbb, _D), lambda i: (i, 0)),
        out_shape=jax.ShapeDtypeStruct((_B, _D), jnp.float32),
    )(hidden, Wv, bv.reshape(1, _D))


def _sc_body(values_hbm, nv_hbm, wp_hbm, out_hbm,
             wp_v, nv_v, buf0, buf1, buf2, buf3, insem, outsem, scsem):
    bufs = (buf0, buf1, buf2, buf3)
    wid = lax.axis_index("s") * 2 + lax.axis_index("c")
    base = wid * _SPW
    rbase = base * _S

    pltpu.sync_copy(wp_hbm.at[pl.ds(base, _SPW)], wp_v)
    pltpu.sync_copy(nv_hbm.at[pl.ds(base, _SPW)], nv_v)

    def copy_group(g, carry):
        hin = []
        for j in range(_CG):
            r0 = rbase + (g * _CG + j) * _S
            hin.append(pltpu.async_copy(
                values_hbm.at[pl.ds(r0, _S)], bufs[j], insem))
        for h in hin:
            h.wait()
        hout = []
        for j in range(_CG):
            r0 = rbase + (g * _CG + j) * _S
            hout.append(pltpu.async_copy(
                bufs[j], out_hbm.at[pl.ds(r0, _S)], outsem))
        for h in hout:
            h.wait()
        return carry

    lax.fori_loop(0, _SPW // _CG, copy_group, 0)

    iota16 = lax.iota(jnp.int32, 16)

    def scatter_group(g, carry):
        wp16 = wp_v[pl.ds(g * 16, 16)]
        rows = (base + g * 16 + iota16) * _S + wp16
        pltpu.async_copy(nv_v.at[pl.ds(g * 16, 16)],
                         out_hbm.at[rows], scsem).wait()
        return carry

    lax.fori_loop(0, _SPW // 16, scatter_group, 0)


def _sc_new_values(values2, nv, wp1):
    mesh = plsc.VectorSubcoreMesh(core_axis_name="c", subcore_axis_name="s")
    fn = functools.partial(
        pl.kernel, mesh=mesh,
        out_type=jax.ShapeDtypeStruct((_B * _S, _D), jnp.float32),
        scratch_types=[
            pltpu.VMEM((_SPW,), jnp.int32),
            pltpu.VMEM((_SPW, _D), jnp.float32),
            pltpu.VMEM((_S, _D), jnp.float32),
            pltpu.VMEM((_S, _D), jnp.float32),
            pltpu.VMEM((_S, _D), jnp.float32),
            pltpu.VMEM((_S, _D), jnp.float32),
            pltpu.SemaphoreType.DMA,
            pltpu.SemaphoreType.DMA,
            pltpu.SemaphoreType.DMA,
        ],
    )(_sc_body)
    return fn(values2, nv, wp1)


def _body(wps_ref, wp_ref, hidden_ref, keys_ref, values_ref, wq_ref, bq_ref,
          wk_ref, bk_ref, wo_ref, bo_ref, nvdep_ref,
          read_ref, nk_ref, nptr_ref):
    del nvdep_ref
    h = hidden_ref[...]
    kb3 = keys_ref[...]
    kb = kb3.reshape(_R, _K)
    vb = values_ref[...]
    wpi = wp_ref[...]

    seg_of_row = jax.lax.broadcasted_iota(jnp.int32, (_R, _BB), 0) // _S
    col = jax.lax.broadcasted_iota(jnp.int32, (_R, _BB), 1)
    E = (seg_of_row == col).astype(jnp.float32)
    seg_of_rowT = jax.lax.broadcasted_iota(jnp.int32, (_BB, _R), 1) // _S
    rowT = jax.lax.broadcasted_iota(jnp.int32, (_BB, _R), 0)
    ET = (seg_of_rowT == rowT).astype(jnp.float32)

    q = _dotT(h, wq_ref[...]) + bq_ref[...]
    q_e = _dot(E, q)
    logits = jnp.sum(kb * q_e, axis=1, keepdims=True)
    e = jnp.exp(logits * np.float32(1.0 / np.sqrt(_K)))
    seg_sum = _dot(ET, e)
    den = _dot(E, seg_sum)
    p = e / den
    readv = _dot(ET, p * vb)
    read_ref[...] = _dotT(readv, wo_ref[...]) + bo_ref[...]

    nk = _dotT(h, wk_ref[...]) + bk_ref[...]
    nk_ref[...] = kb3
    for i in range(_BB):
        nk_ref[i, pl.ds(wps_ref[i, 0], 1), :] = nk[i:i + 1, :]
    nptr_ref[...] = (wpi + 1) % _S


def kernel(hidden, keys, values, write_ptr, Wq, bq, Wk, bk, Wv, bv, Wo, bo):
    nb = _B // _BB
    wp1 = write_ptr.astype(jnp.int32)
    wp2 = wp1.reshape(_B, 1)
    values2 = values.reshape(_B * _S, _D)

    nv = _project_nv(hidden, Wv, bv)
    new_values2 = _sc_new_values(values2, nv, wp1)

    full = lambda shp: pl.BlockSpec(shp, lambda i: (0,) * len(shp))
    out = pl.pallas_call(
        _body,
        grid=(nb,),
        in_specs=[
            pl.BlockSpec((_BB, 1), lambda i: (i, 0),
                         memory_space=pltpu.SMEM),
            pl.BlockSpec((_BB, 1), lambda i: (i, 0)),
            pl.BlockSpec((_BB, _D), lambda i: (i, 0)),
            pl.BlockSpec((_BB, _S, _K), lambda i: (i, 0, 0)),
            pl.BlockSpec((_R, _D), lambda i: (i, 0)),
            full((_K, _D)), full((1, _K)),
            full((_K, _D)), full((1, _K)),
            full((_D, _D)), full((1, _D)),
            full((1, _D)),
        ],
        out_specs=[
            pl.BlockSpec((_BB, _D), lambda i: (i, 0)),
            pl.BlockSpec((_BB, _S, _K), lambda i: (i, 0, 0)),
            pl.BlockSpec((_BB, 1), lambda i: (i, 0)),
        ],
        out_shape=[
            jax.ShapeDtypeStruct((_B, _D), jnp.float32),
            jax.ShapeDtypeStruct((_B, _S, _K), jnp.float32),
            jax.ShapeDtypeStruct((_B, 1), jnp.int32),
        ],
        compiler_params=pltpu.CompilerParams(
            dimension_semantics=("arbitrary",)),
    )(wp2, wp2, hidden, keys, values2, Wq, bq.reshape(1, _K), Wk,
      bk.reshape(1, _K), Wo, bo.reshape(1, _D), nv[:1])
    read, new_keys, nptr = out
    return (read, new_keys, new_values2.reshape(_B, _S, _D),
            nptr.reshape(_B))


# speedup vs baseline: 1.2842x; 1.2842x over previous
import jax
import jax.numpy as jnp
import numpy as np
from jax.experimental import pallas as pl
from jax.experimental.pallas import tpu as pltpu

_B, _D, _S, _K = 4096, 256, 64, 64
_BB = 64
_R = _BB * _S


def _dot(x, w, prec=jax.lax.Precision.DEFAULT):
    return jax.lax.dot_general(
        x, w, (((1,), (0,)), ((), ())),
        precision=prec, preferred_element_type=jnp.float32)


def _dotT(x, w):
    return jax.lax.dot_general(
        x, w, (((1,), (1,)), ((), ())),
        precision=jax.lax.Precision.HIGHEST,
        preferred_element_type=jnp.float32)


def _keys_body(wps_ref, wp_ref, hidden_ref, keys_ref, wk_ref, bk_ref,
               nk_ref, nptr_ref):
    h = hidden_ref[...]
    kb = keys_ref[...]
    nk = _dotT(h, wk_ref[...]) + bk_ref[...]
    nk_ref[...] = kb
    for i in range(_BB):
        nk_ref[pl.ds(i * _S + wps_ref[i, 0], 1), :] = nk[i:i + 1, :]
    nptr_ref[...] = (wp_ref[...] + 1) % _S


def _values_body(wps_ref, hidden_ref, keys_ref, values_ref, wq_ref, bq_ref,
                 wv_ref, bv_ref, wo_ref, bo_ref, dep_ref,
                 read_ref, nv_ref):
    del dep_ref
    h = hidden_ref[...]
    kb = keys_ref[...]
    vb = values_ref[...]

    seg_of_row = jax.lax.broadcasted_iota(jnp.int32, (_R, _BB), 0) // _S
    col = jax.lax.broadcasted_iota(jnp.int32, (_R, _BB), 1)
    E = (seg_of_row == col).astype(jnp.float32)
    seg_of_rowT = jax.lax.broadcasted_iota(jnp.int32, (_BB, _R), 1) // _S
    rowT = jax.lax.broadcasted_iota(jnp.int32, (_BB, _R), 0)
    ET = (seg_of_rowT == rowT).astype(jnp.float32)

    q = _dotT(h, wq_ref[...]) + bq_ref[...]
    q_e = _dot(E, q)
    logits = jnp.sum(kb * q_e, axis=1, keepdims=True)
    e = jnp.exp(logits * np.float32(1.0 / np.sqrt(_K)))
    seg_sum = _dot(ET, e)
    den = _dot(E, seg_sum)
    p = e / den
    readv = _dot(ET, p * vb)
    read_ref[...] = _dotT(readv, wo_ref[...]) + bo_ref[...]

    nv = _dotT(h, wv_ref[...]) + bv_ref[...]
    nv_ref[...] = vb
    for i in range(_BB):
        nv_ref[pl.ds(i * _S + wps_ref[i, 0], 1), :] = nv[i:i + 1, :]


def kernel(hidden, keys, values, write_ptr, Wq, bq, Wk, bk, Wv, bv, Wo, bo):
    nb = _B // _BB
    wp2 = write_ptr.astype(jnp.int32).reshape(_B, 1)
    keys2 = keys.reshape(_B * _S, _K)
    values2 = values.reshape(_B * _S, _D)
    full = lambda shp: pl.BlockSpec(shp, lambda i: (0,) * len(shp))
    smem = lambda: pl.BlockSpec((_BB, 1), lambda i: (i, 0),
                                memory_space=pltpu.SMEM)

    nk2, nptr = pl.pallas_call(
        _keys_body,
        grid=(nb,),
        in_specs=[
            smem(),
            pl.BlockSpec((_BB, 1), lambda i: (i, 0)),
            pl.BlockSpec((_BB, _D), lambda i: (i, 0)),
            pl.BlockSpec((_R, _K), lambda i: (i, 0)),
            full((_K, _D)), full((1, _K)),
        ],
        out_specs=[
            pl.BlockSpec((_R, _K), lambda i: (i, 0)),
            pl.BlockSpec((_BB, 1), lambda i: (i, 0)),
        ],
        out_shape=[
            jax.ShapeDtypeStruct((_B * _S, _K), jnp.float32),
            jax.ShapeDtypeStruct((_B, 1), jnp.int32),
        ],
        compiler_params=pltpu.CompilerParams(
            dimension_semantics=("arbitrary",)),
    )(wp2, wp2, hidden, keys2, Wk, bk.reshape(1, _K))

    read, nv2 = pl.pallas_call(
        _values_body,
        grid=(nb,),
        in_specs=[
            smem(),
            pl.BlockSpec((_BB, _D), lambda i: (i, 0)),
            pl.BlockSpec((_R, _K), lambda i: (i, 0)),
            pl.BlockSpec((_R, _D), lambda i: (i, 0)),
            full((_K, _D)), full((1, _K)),
            full((_D, _D)), full((1, _D)),
            full((_D, _D)), full((1, _D)),
            full((1, _K)),
        ],
        out_specs=[
            pl.BlockSpec((_BB, _D), lambda i: (i, 0)),
            pl.BlockSpec((_R, _D), lambda i: (i, 0)),
        ],
        out_shape=[
            jax.ShapeDtypeStruct((_B, _D), jnp.float32),
            jax.ShapeDtypeStruct((_B * _S, _D), jnp.float32),
        ],
        compiler_params=pltpu.CompilerParams(
            dimension_semantics=("arbitrary",)),
    )(wp2, hidden, keys2, values2, Wq, bq.reshape(1, _K), Wv,
      bv.reshape(1, _D), Wo, bo.reshape(1, _D), nk2[:1])

    return (read, nk2.reshape(_B, _S, _K), nv2.reshape(_B, _S, _D),
            nptr.reshape(_B))


# speedup vs baseline: 1.4501x; 1.1292x over previous
import jax
import jax.numpy as jnp
import numpy as np
from jax.experimental import pallas as pl
from jax.experimental.pallas import tpu as pltpu

_B, _D, _S, _K = 4096, 256, 64, 64
_BB = 64
_R = _BB * _S


def _dot(x, w, prec=jax.lax.Precision.DEFAULT):
    return jax.lax.dot_general(
        x, w, (((1,), (0,)), ((), ())),
        precision=prec, preferred_element_type=jnp.float32)


def _dotT(x, w):
    return jax.lax.dot_general(
        x, w, (((1,), (1,)), ((), ())),
        precision=jax.lax.Precision.HIGHEST,
        preferred_element_type=jnp.float32)


def _body(wps_ref, wp_ref, hidden_ref, keys_ref, values_ref, wq_ref, bq_ref,
          wk_ref, bk_ref, wv_ref, bv_ref, wo_ref, bo_ref,
          read_ref, nk_ref, nv_ref, nptr_ref):
    h = hidden_ref[...]
    kb = keys_ref[...]
    vb = values_ref[...]
    wpi = wp_ref[...]

    seg_of_row = jax.lax.broadcasted_iota(jnp.int32, (_R, _BB), 0) // _S
    col = jax.lax.broadcasted_iota(jnp.int32, (_R, _BB), 1)
    E = (seg_of_row == col).astype(jnp.float32)
    seg_of_rowT = jax.lax.broadcasted_iota(jnp.int32, (_BB, _R), 1) // _S
    rowT = jax.lax.broadcasted_iota(jnp.int32, (_BB, _R), 0)
    ET = (seg_of_rowT == rowT).astype(jnp.float32)

    q = _dotT(h, wq_ref[...]) + bq_ref[...]
    q_e = _dot(E, q)
    logits = jnp.sum(kb * q_e, axis=1, keepdims=True)
    e = jnp.exp(logits * np.float32(1.0 / np.sqrt(_K)))
    seg_sum = _dot(ET, e)
    den = _dot(E, seg_sum)
    p = e / den
    readv = _dot(ET, p * vb)
    read_ref[...] = _dotT(readv, wo_ref[...]) + bo_ref[...]

    nk = _dotT(h, wk_ref[...]) + bk_ref[...]
    nv = _dotT(h, wv_ref[...]) + bv_ref[...]
    nk_ref[...] = kb
    nv_ref[...] = vb
    for i in range(_BB):
        base = i * _S + wps_ref[i, 0]
        nk_ref[pl.ds(base, 1), :] = nk[i:i + 1, :]
        nv_ref[pl.ds(base, 1), :] = nv[i:i + 1, :]
    nptr_ref[...] = (wpi + 1) % _S


def kernel(hidden, keys, values, write_ptr, Wq, bq, Wk, bk, Wv, bv, Wo, bo):
    nb = _B // _BB
    wp2 = write_ptr.astype(jnp.int32).reshape(_B, 1)
    keys2 = keys.reshape(_B * _S, _K)
    values2 = values.reshape(_B * _S, _D)
    full = lambda shp: pl.BlockSpec(shp, lambda i: (0,) * len(shp))
    out = pl.pallas_call(
        _body,
        grid=(nb,),
        in_specs=[
            pl.BlockSpec((_BB, 1), lambda i: (i, 0),
                         memory_space=pltpu.SMEM),
            pl.BlockSpec((_BB, 1), lambda i: (i, 0)),
            pl.BlockSpec((_BB, _D), lambda i: (i, 0)),
            pl.BlockSpec((_R, _K), lambda i: (i, 0)),
            pl.BlockSpec((_R, _D), lambda i: (i, 0)),
            full((_K, _D)), full((1, _K)),
            full((_K, _D)), full((1, _K)),
            full((_D, _D)), full((1, _D)),
            full((_D, _D)), full((1, _D)),
        ],
        out_specs=[
            pl.BlockSpec((_BB, _D), lambda i: (i, 0)),
            pl.BlockSpec((_R, _K), lambda i: (i, 0)),
            pl.BlockSpec((_R, _D), lambda i: (i, 0)),
            pl.BlockSpec((_BB, 1), lambda i: (i, 0)),
        ],
        out_shape=[
            jax.ShapeDtypeStruct((_B, _D), jnp.float32),
            jax.ShapeDtypeStruct((_B * _S, _K), jnp.float32),
            jax.ShapeDtypeStruct((_B * _S, _D), jnp.float32),
            jax.ShapeDtypeStruct((_B, 1), jnp.int32),
        ],
        compiler_params=pltpu.CompilerParams(
            dimension_semantics=("arbitrary",)),
    )(wp2, wp2, hidden, keys2, values2, Wq, bq.reshape(1, _K), Wk,
      bk.reshape(1, _K), Wv, bv.reshape(1, _D), Wo, bo.reshape(1, _D))
    read, nk2, nv2, nptr = out
    return (read, nk2.reshape(_B, _S, _K), nv2.reshape(_B, _S, _D),
            nptr.reshape(_B))
